# Initial kernel scaffold; baseline (speedup 1.0000x reference)
import jax
import jax.numpy as jnp
from jax.experimental import pallas as pl


def kernel(centers_a, centers_b):
    raise NotImplementedError("write your pallas kernel here")


# speedup vs baseline: 17.8020x; 17.8020x over previous
import jax
import jax.numpy as jnp
from jax.experimental import pallas as pl
from jax.experimental.pallas import tpu as pltpu

_GAMMA = 0.001
_NEG = -1e30
_M = 2048
_N = 2048
_D = 512
_RB = 256
_R2 = 256
_LANES = _N // 128


def _plse(a, b):
    m = jnp.maximum(a, b)
    return m + jnp.log1p(jnp.exp(-jnp.abs(a - b)))


def _cost_kernel(a_ref, b_ref, e_ref):
    a = a_ref[...]
    b = b_ref[...]
    ra = jax.lax.rsqrt(jnp.sum(a * a, axis=1, keepdims=True) + 1e-10)
    rb = jax.lax.rsqrt(jnp.sum(b * b, axis=1, keepdims=True) + 1e-10)
    d = jax.lax.dot_general(a * ra, b * rb, (((1,), (1,)), ((), ())),
                            preferred_element_type=jnp.float32)
    e = jnp.exp(-1.0 - d)
    for c in range(_LANES):
        e_ref[:, c, :] = e[:, c * 128:(c + 1) * 128]


def _dp_kernel(e_ref, o_ref, p_ref, plast_ref):
    @pl.when(pl.program_id(0) == 0)
    def _():
        p_ref[...] = jnp.zeros((_LANES, 128), jnp.float32)
        plast_ref[...] = jnp.zeros((1, 1), jnp.float32)

    def body(r, carry):
        e16 = e_ref[r]
        sig = jnp.sum(e16, axis=(0, 1), keepdims=True)
        su = 1000.0 / (sig + 2049.0)
        ru = e16 * su
        p = p_ref[...]
        w = -jnp.minimum(su, ru) + jnp.log1p(jnp.exp(-jnp.abs(su - ru)))
        x = p + w
        for sh in (1, 2, 4, 8, 16, 32, 64):
            sx = jnp.concatenate(
                [jnp.full((_LANES, sh), _NEG, jnp.float32), x[:, :-sh]], axis=1)
            x = _plse(x, sx)
        y = x[:, 127:128]
        for sh in (1, 2, 4, 8):
            sy = jnp.concatenate(
                [jnp.full((sh, 1), _NEG, jnp.float32), y[:-sh, :]], axis=0)
            y = _plse(y, sy)
        carry16 = jnp.concatenate(
            [jnp.full((1, 1), _NEG, jnp.float32), y[:-1, :]], axis=0)
        cum = _plse(x, carry16)
        pcol = jnp.concatenate(
            [jnp.full((1, 1), _NEG, jnp.float32), cum[:-1, 127:128]], axis=0)
        scum = jnp.concatenate([pcol, cum[:, :-1]], axis=1)
        p_ref[...] = _plse(scum, p - su)
        plast_ref[...] = _plse(cum[15:16, 127:128], plast_ref[...] - su)
        return carry

    jax.lax.fori_loop(0, _R2, body, 0)
    o_ref[...] = -_GAMMA * plast_ref[...]


def _build_calls(interpret=False):
    cost = pl.pallas_call(
        _cost_kernel,
        grid=(_M // _RB,),
        in_specs=[pl.BlockSpec((_RB, _D), lambda i: (i, 0)),
                  pl.BlockSpec((_N, _D), lambda i: (0, 0))],
        out_specs=pl.BlockSpec((_RB, _LANES, 128), lambda i: (i, 0, 0)),
        out_shape=jax.ShapeDtypeStruct((_M, _LANES, 128), jnp.float32),
        compiler_params=pltpu.CompilerParams(
            dimension_semantics=("parallel",)),
        interpret=interpret,
    )
    dp = pl.pallas_call(
        _dp_kernel,
        grid=(_M // _R2,),
        in_specs=[pl.BlockSpec((_R2, _LANES, 128), lambda i: (i, 0, 0))],
        out_specs=pl.BlockSpec((1, 1), lambda i: (0, 0)),
        out_shape=jax.ShapeDtypeStruct((1, 1), jnp.float32),
        scratch_shapes=[pltpu.VMEM((_LANES, 128), jnp.float32),
                        pltpu.VMEM((1, 1), jnp.float32)],
        compiler_params=pltpu.CompilerParams(
            dimension_semantics=("arbitrary",)),
        interpret=interpret,
    )
    return cost, dp


def kernel(centers_a, centers_b):
    a = centers_a[jnp.argsort(centers_a[:, -1])][:, :-1]
    b = centers_b[jnp.argsort(centers_b[:, -1])][:, :-1]
    cost, dp = _build_calls()
    return dp(cost(a, b))[0, 0]


# speedup vs baseline: 48.0755x; 2.7006x over previous
import functools

import jax
import jax.numpy as jnp
from jax.experimental import pallas as pl
from jax.experimental.pallas import tpu as pltpu

_GAMMA = 0.001
_NEG = -1e30
_M = 2048
_N = 2048
_D = 512
_RB = 256
_R2 = 256
_LANES = _N // 128


def _plse(a, b):
    m = jnp.maximum(a, b)
    return m + jnp.log1p(jnp.exp(-jnp.abs(a - b)))


def _lse_many(xs):
    m = functools.reduce(jnp.maximum, xs)
    s = functools.reduce(lambda acc, x: acc + x, [jnp.exp(x - m) for x in xs])
    return m + jnp.log(s)


def _shift_lanes(x, sh):
    fill = jnp.full(x.shape[:-1] + (sh,), _NEG, jnp.float32)
    return jnp.concatenate([fill, x[..., :-sh]], axis=-1)


def _shift_rows(x, sh):
    fill = jnp.full((sh,) + x.shape[1:], _NEG, jnp.float32)
    return jnp.concatenate([fill, x[:-sh, ...]], axis=0)


def _cost_kernel(a_ref, b_ref, e_ref, su_ref):
    a = a_ref[...]
    b = b_ref[...]
    ra = jax.lax.rsqrt(jnp.sum(a * a, axis=1, keepdims=True) + 1e-10)
    rb = jax.lax.rsqrt(jnp.sum(b * b, axis=1, keepdims=True) + 1e-10)
    d = jax.lax.dot_general(a * ra, b * rb, (((1,), (1,)), ((), ())),
                            preferred_element_type=jnp.float32)
    e = jnp.exp(-1.0 - d)
    for c in range(_LANES):
        e_ref[:, c, :] = e[:, c * 128:(c + 1) * 128]
    su_ref[...] = 1000.0 / (jnp.sum(e, axis=1, keepdims=True) + float(_N + 1))


def _dp_kernel(e_ref, su_ref, o_ref, p_ref, plast_ref):
    @pl.when(pl.program_id(0) == 0)
    def _():
        p_ref[...] = jnp.zeros((_LANES, 128), jnp.float32)
        plast_ref[...] = jnp.zeros((1, 1), jnp.float32)

    def body(r, carry):
        su = su_ref[r, 0]
        e16 = e_ref[r]
        ru = e16 * su
        p = p_ref[...]
        w = -jnp.minimum(su, ru) + jnp.log1p(jnp.exp(-jnp.abs(su - ru)))
        x = p + w
        x = _lse_many([x] + [_shift_lanes(x, sh) for sh in range(1, 16)])
        x = _lse_many([x] + [_shift_lanes(x, 16 * q) for q in range(1, 8)])
        rt = x[:, 127:128]
        carry16 = _lse_many([_shift_rows(rt, m) for m in range(1, 16)])
        cum = _plse(x, carry16)
        pcol = _shift_rows(cum[:, 127:128], 1)
        scum = jnp.concatenate([pcol, cum[:, :-1]], axis=1)
        p_ref[...] = _plse(scum, p - su)
        plast_ref[...] = _plse(cum[15:16, 127:128], plast_ref[...] - su)
        return carry

    jax.lax.fori_loop(0, _R2, body, 0)
    o_ref[...] = -_GAMMA * plast_ref[...]


def _build_calls(interpret=False):
    cost = pl.pallas_call(
        _cost_kernel,
        grid=(_M // _RB,),
        in_specs=[pl.BlockSpec((_RB, _D), lambda i: (i, 0)),
                  pl.BlockSpec((_N, _D), lambda i: (0, 0))],
        out_specs=(pl.BlockSpec((_RB, _LANES, 128), lambda i: (i, 0, 0)),
                   pl.BlockSpec((_RB, 1), lambda i: (i, 0))),
        out_shape=(jax.ShapeDtypeStruct((_M, _LANES, 128), jnp.float32),
                   jax.ShapeDtypeStruct((_M, 1), jnp.float32)),
        compiler_params=pltpu.CompilerParams(
            dimension_semantics=("parallel",)),
        interpret=interpret,
    )
    dp = pl.pallas_call(
        _dp_kernel,
        grid=(_M // _R2,),
        in_specs=[pl.BlockSpec((_R2, _LANES, 128), lambda i: (i, 0, 0)),
                  pl.BlockSpec((_R2, 1), lambda i: (i, 0),
                               memory_space=pltpu.SMEM)],
        out_specs=pl.BlockSpec((1, 1), lambda i: (0, 0)),
        out_shape=jax.ShapeDtypeStruct((1, 1), jnp.float32),
        scratch_shapes=[pltpu.VMEM((_LANES, 128), jnp.float32),
                        pltpu.VMEM((1, 1), jnp.float32)],
        compiler_params=pltpu.CompilerParams(
            dimension_semantics=("arbitrary",)),
        interpret=interpret,
    )
    return cost, dp


def kernel(centers_a, centers_b):
    a = centers_a[jnp.argsort(centers_a[:, -1])][:, :-1]
    b = centers_b[jnp.argsort(centers_b[:, -1])][:, :-1]
    cost, dp = _build_calls()
    e, su = cost(a, b)
    return dp(e, su)[0, 0]
